# Initial kernel scaffold; baseline (speedup 1.0000x reference)
import jax
import jax.numpy as jnp
from jax.experimental import pallas as pl


def kernel(input_batch, W_router, W_in, W_out):
    raise NotImplementedError("write your pallas kernel here")


# speedup vs baseline: 1.2915x; 1.2915x over previous
import functools

import jax
import jax.numpy as jnp
from jax.experimental import pallas as pl

E = 8
TOP_K = 2
D_MODEL = 768
D_FF = 3072
FF_BLK = 768


def _router_kernel(x_ref, wr_ref, gates_ref):
    x = x_ref[...]
    logits = jnp.dot(x, wr_ref[...], preferred_element_type=jnp.float32)
    m = jnp.max(logits, axis=-1, keepdims=True)
    ex = jnp.exp(logits - m)
    probs = ex / jnp.sum(ex, axis=-1, keepdims=True)
    iota = jax.lax.broadcasted_iota(jnp.int32, probs.shape, 1)
    p1 = jnp.max(probs, axis=-1, keepdims=True)
    i1 = jnp.min(jnp.where(probs == p1, iota, E), axis=-1, keepdims=True)
    probs2 = jnp.where(iota == i1, -jnp.inf, probs)
    p2 = jnp.max(probs2, axis=-1, keepdims=True)
    i2 = jnp.min(jnp.where(probs2 == p2, iota, E), axis=-1, keepdims=True)
    gates = jnp.where(iota == i1, p1, jnp.where(iota == i2, p2, 0.0))
    gates_ref[...] = gates


def _ffn_kernel(x_ref, gates_ref, win_ref, wout_ref, out_ref):
    e = pl.program_id(0)
    f = pl.program_id(1)

    @pl.when((e == 0) & (f == 0))
    def _():
        out_ref[...] = jnp.zeros_like(out_ref)

    h = jnp.dot(x_ref[...], win_ref[0], preferred_element_type=jnp.float32)
    h = jnp.maximum(h, 0.0)
    y = jnp.dot(h, wout_ref[0], preferred_element_type=jnp.float32)
    gates = gates_ref[...]
    col = jax.lax.broadcasted_iota(jnp.int32, gates.shape, 1)
    g = jnp.sum(jnp.where(col == e, gates, 0.0), axis=1, keepdims=True)
    out_ref[...] += g * y


@jax.jit
def kernel(input_batch, W_router, W_in, W_out):
    b, s, d = input_batch.shape
    x = input_batch.reshape(-1, d)
    T = x.shape[0]

    gates = pl.pallas_call(
        _router_kernel,
        out_shape=jax.ShapeDtypeStruct((T, E), jnp.float32),
    )(x, W_router)

    n_ff = D_FF // FF_BLK
    out = pl.pallas_call(
        _ffn_kernel,
        grid=(E, n_ff),
        in_specs=[
            pl.BlockSpec((T, D_MODEL), lambda e, f: (0, 0)),
            pl.BlockSpec((T, E), lambda e, f: (0, 0)),
            pl.BlockSpec((1, D_MODEL, FF_BLK), lambda e, f: (e, 0, f)),
            pl.BlockSpec((1, FF_BLK, D_MODEL), lambda e, f: (e, f, 0)),
        ],
        out_specs=pl.BlockSpec((T, D_MODEL), lambda e, f: (0, 0)),
        out_shape=jax.ShapeDtypeStruct((T, D_MODEL), jnp.float32),
    )(x, gates, W_in, W_out)

    return out.reshape(b, s, d)


# speedup vs baseline: 1.3014x; 1.0076x over previous
import functools

import jax
import jax.numpy as jnp
from jax.experimental import pallas as pl

E = 8
TOP_K = 2
D_MODEL = 768
D_FF = 3072
FF_BLK = 768


def _router_kernel(x_ref, wr_ref, gates_ref):
    x = x_ref[...]
    logits = jnp.dot(x, wr_ref[...], preferred_element_type=jnp.float32)
    m = jnp.max(logits, axis=-1, keepdims=True)
    ex = jnp.exp(logits - m)
    probs = ex / jnp.sum(ex, axis=-1, keepdims=True)
    iota = jax.lax.broadcasted_iota(jnp.int32, probs.shape, 1)
    p1 = jnp.max(probs, axis=-1, keepdims=True)
    i1 = jnp.min(jnp.where(probs == p1, iota, E), axis=-1, keepdims=True)
    probs2 = jnp.where(iota == i1, -jnp.inf, probs)
    p2 = jnp.max(probs2, axis=-1, keepdims=True)
    i2 = jnp.min(jnp.where(probs2 == p2, iota, E), axis=-1, keepdims=True)
    gates = jnp.where(iota == i1, p1, jnp.where(iota == i2, p2, 0.0))
    gates_ref[...] = gates


def _ffn_kernel(x_ref, gates_ref, win_ref, wout_ref, out_ref):
    e = pl.program_id(0)
    f = pl.program_id(1)

    @pl.when((e == 0) & (f == 0))
    def _():
        out_ref[...] = jnp.zeros_like(out_ref)

    h = jnp.dot(x_ref[...].astype(jnp.bfloat16), win_ref[0].astype(jnp.bfloat16),
                preferred_element_type=jnp.float32)
    h = jnp.maximum(h, 0.0)
    y = jnp.dot(h.astype(jnp.bfloat16), wout_ref[0].astype(jnp.bfloat16),
                preferred_element_type=jnp.float32)
    gates = gates_ref[...]
    col = jax.lax.broadcasted_iota(jnp.int32, gates.shape, 1)
    g = jnp.sum(jnp.where(col == e, gates, 0.0), axis=1, keepdims=True)
    out_ref[...] += g * y


@jax.jit
def kernel(input_batch, W_router, W_in, W_out):
    b, s, d = input_batch.shape
    x = input_batch.reshape(-1, d)
    T = x.shape[0]

    gates = pl.pallas_call(
        _router_kernel,
        out_shape=jax.ShapeDtypeStruct((T, E), jnp.float32),
    )(x, W_router)

    n_ff = D_FF // FF_BLK
    out = pl.pallas_call(
        _ffn_kernel,
        grid=(E, n_ff),
        in_specs=[
            pl.BlockSpec((T, D_MODEL), lambda e, f: (0, 0)),
            pl.BlockSpec((T, E), lambda e, f: (0, 0)),
            pl.BlockSpec((1, D_MODEL, FF_BLK), lambda e, f: (e, 0, f)),
            pl.BlockSpec((1, FF_BLK, D_MODEL), lambda e, f: (e, f, 0)),
        ],
        out_specs=pl.BlockSpec((T, D_MODEL), lambda e, f: (0, 0)),
        out_shape=jax.ShapeDtypeStruct((T, D_MODEL), jnp.float32),
    )(x, gates, W_in, W_out)

    return out.reshape(b, s, d)
